# Initial kernel scaffold; baseline (speedup 1.0000x reference)
import jax
import jax.numpy as jnp
from jax.experimental import pallas as pl


def kernel(x, Wl1, Wr1, att1, bconv1, g1, be1, Wl2, Wr2, att2, bconv2, g2, be2, Wl3, Wr3, att3, bconv3, g3, be3, cW1, cb1, cW2, cb2, edge_index, batch):
    raise NotImplementedError("write your pallas kernel here")


# speedup vs baseline: 18.2437x; 18.2437x over previous
import functools

import jax
import jax.numpy as jnp
from jax import lax
from jax.experimental import pallas as pl
from jax.experimental.pallas import tpu as pltpu
from jax.experimental.pallas import tpu_sc as plsc

N = 10000
E = 320000
G = 64
HID = 64

_NSUB = 16
_B = 128
_NBLK = E // _B
_RPW = 624
_DROW = 157

_mesh = plsc.VectorSubcoreMesh(core_axis_name="c", subcore_axis_name="s")
_sc_params = pltpu.CompilerParams(needs_layout_passes=False)


def _zero_rows(zbuf, acc, s):
    nz = zbuf.shape[0]
    @pl.loop(0, nz)
    def _(i):
        for k in range(8):
            zbuf[i, pl.ds(k * 16, 16)] = jnp.zeros((16,), jnp.float32)
    row0 = s * _RPW
    full = (_RPW // nz) * nz
    rem = _RPW - full

    @pl.loop(0, full, step=nz)
    def _(r):
        pltpu.sync_copy(zbuf, acc.at[pl.ds(row0 + r, nz)])
    if rem:
        pltpu.sync_copy(zbuf.at[pl.ds(0, rem)],
                        acc.at[pl.ds(row0 + full, rem)])

    @pl.when(s == _NSUB - 1)
    def _():
        pltpu.sync_copy(zbuf.at[pl.ds(0, 16)],
                        acc.at[pl.ds(_NSUB * _RPW, 16)])


def _copy_out(buf, acc, out, c, s):
    nz = buf.shape[0]
    row0 = s * _RPW
    full = (_RPW // nz) * nz
    rem = _RPW - full

    @pl.loop(0, full, step=nz)
    def _(r):
        pltpu.sync_copy(acc.at[pl.ds(row0 + r, nz)], buf)
        pltpu.sync_copy(buf, out.at[c, pl.ds(row0 + r, nz)])
    if rem:
        pltpu.sync_copy(acc.at[pl.ds(row0 + full, rem)], buf.at[pl.ds(0, rem)])
        pltpu.sync_copy(buf.at[pl.ds(0, rem)],
                        out.at[c, pl.ds(row0 + full, rem)])

    @pl.when(s == _NSUB - 1)
    def _():
        pltpu.sync_copy(acc.at[pl.ds(_NSUB * _RPW, 16)], buf.at[pl.ds(0, 16)])
        pltpu.sync_copy(buf.at[pl.ds(0, 16)], out.at[c, pl.ds(_NSUB * _RPW, 16)])


def _edge_phase_4h(xl_cat, xr_cat, src, dst, att):
    @functools.partial(
        pl.kernel,
        out_type=[jax.ShapeDtypeStruct((2, N, 128), jnp.float32),
                  jax.ShapeDtypeStruct((2, _NSUB, _DROW, 128), jnp.float32)],
        mesh=_mesh,
        compiler_params=_sc_params,
        scratch_types=[
            pltpu.VMEM_SHARED((N, 128), jnp.float32),
            pltpu.VMEM((_DROW, 128), jnp.float32),
            pltpu.VMEM((4, 64), jnp.float32),
            pltpu.VMEM((1, _B), jnp.int32),
            pltpu.VMEM((1, _B), jnp.int32),
            pltpu.VMEM((1, 64), jnp.int32),
            pltpu.VMEM((1, 64), jnp.int32),
            pltpu.VMEM((1, 64), jnp.int32),
            pltpu.VMEM((1, 80), jnp.int32),
            pltpu.VMEM((64, 128), jnp.float32),
            pltpu.VMEM((64, 128), jnp.float32),
            pltpu.VMEM((64, 128), jnp.float32),
        ],
    )
    def k(xl_hbm, xr_hbm, src_hbm, dst_hbm, att_hbm, num_hbm, denp_hbm,
          acc, denp, attv, sidx, didx, s64, d64, d64cn, didxe,
          xlb, xrb, msgb):
        c = lax.axis_index("c")
        s = lax.axis_index("s")
        _zero_rows(msgb, acc, s)

        @pl.loop(0, _DROW)
        def _(i):
            for k_ in range(8):
                denp[i, pl.ds(k_ * 16, 16)] = jnp.zeros((16,), jnp.float32)

        pltpu.sync_copy(att_hbm, attv)
        plsc.subcore_barrier()

        cn = c * N
        iota = lax.iota(jnp.int32, 16)
        attr = [[plsc.load_gather(
                    attv,
                    [jnp.broadcast_to(2 * c + h, (16,)).astype(jnp.int32),
                     k4 * 16 + iota])
                 for k4 in range(4)] for h in range(2)]

        nb = jnp.where(s < _NBLK % _NSUB, _NBLK // _NSUB + 1, _NBLK // _NSUB)

        @pl.loop(0, nb)
        def _(j):
            e = (s + _NSUB * j) * _B
            pltpu.sync_copy(src_hbm.at[:, pl.ds(e, _B)], sidx)
            pltpu.sync_copy(dst_hbm.at[:, pl.ds(e, _B)], didx)
            for half in range(2):
                for t in range(0, 64, 16):
                    sv = sidx[0, pl.ds(half * 64 + t, 16)]
                    dv = didx[0, pl.ds(half * 64 + t, 16)]
                    s64[0, pl.ds(t, 16)] = sv + cn
                    d64[0, pl.ds(t, 16)] = dv
                    d64cn[0, pl.ds(t, 16)] = dv + cn
                    didxe[0, pl.ds(t, 16)] = dv
                pltpu.sync_copy(xl_hbm.at[s64.at[0]], xlb)
                pltpu.sync_copy(xr_hbm.at[d64cn.at[0]], xrb)

                _edge_body_4h(attv, didxe, xlb, xrb, msgb, denp, attr, iota)
                pltpu.sync_copy(msgb, acc.at[d64.at[0]], add=True)

        pltpu.sync_copy(denp, denp_hbm.at[c, s])
        plsc.subcore_barrier()
        _copy_out(msgb, acc, num_hbm, c, s)

    return k(xl_cat, xr_cat, src, dst, att)


def _edge_body_4h(attv, didxe, xlb, xrb, msgb, denp, attr, iota):
    @pl.loop(0, 64)
    def _(i):
                xl = [xlb[i, pl.ds(k_ * 16, 16)] for k_ in range(8)]
                exs = []
                for h in range(2):
                    acc_v = jnp.zeros((16,), jnp.float32)
                    for k4 in range(4):
                        k_ = h * 4 + k4
                        ev = xl[k_] + xrb[i, pl.ds(k_ * 16, 16)]
                        ea = jnp.maximum(ev, 0.2 * ev)
                        acc_v = acc_v + ea * attr[h][k4]
                    alpha = jnp.sum(acc_v)
                    exv = jnp.exp(jnp.broadcast_to(alpha, (16,)))
                    exs.append(exv)
                    for k4 in range(4):
                        k_ = h * 4 + k4
                        msgb[i, pl.ds(k_ * 16, 16)] = xl[k_] * exv
                d = didxe[0, pl.ds(i, 16)][0]
                rowv = jnp.broadcast_to(d >> 6, (16,)).astype(jnp.int32)
                lanev = jnp.broadcast_to((d & 63) * 2, (16,)).astype(
                    jnp.int32) + iota
                vals = jnp.where(iota == 0, exs[0], exs[1])
                plsc.addupdate_scatter(denp, [rowv, lanev], vals,
                                       mask=iota < 2)


def _edge_phase_1h(xl3, xr3, src, dst, att):
    @functools.partial(
        pl.kernel,
        out_type=jax.ShapeDtypeStruct((2, N, 128), jnp.float32),
        mesh=_mesh,
        compiler_params=_sc_params,
        scratch_types=[
            pltpu.VMEM_SHARED((N, 128), jnp.float32),
            pltpu.VMEM((1, 64), jnp.float32),
            pltpu.VMEM((1, _B), jnp.int32),
            pltpu.VMEM((1, _B), jnp.int32),
            pltpu.VMEM((1, 64), jnp.int32),
            pltpu.VMEM((1, 64), jnp.int32),
            pltpu.VMEM((64, 128), jnp.float32),
            pltpu.VMEM((64, 128), jnp.float32),
            pltpu.VMEM((64, 128), jnp.float32),
        ],
    )
    def k(xl_hbm, xr_hbm, src_hbm, dst_hbm, att_hbm, out_hbm,
          acc, attv, sidx, didx, s64, d64, xlb, xrb, msgb):
        c = lax.axis_index("c")
        s = lax.axis_index("s")
        _zero_rows(msgb, acc, s)
        pltpu.sync_copy(att_hbm, attv)
        plsc.subcore_barrier()

        iota = lax.iota(jnp.int32, 16)
        zero16 = jnp.broadcast_to(0, (16,)).astype(jnp.int32)
        attr = [plsc.load_gather(attv, [zero16, k4 * 16 + iota])
                for k4 in range(4)]

        w = c * _NSUB + s
        nw = 2 * _NSUB
        nb = jnp.where(w < _NBLK % nw, _NBLK // nw + 1, _NBLK // nw)

        @pl.loop(0, nb)
        def _(j):
            e = (w + nw * j) * _B
            pltpu.sync_copy(src_hbm.at[:, pl.ds(e, _B)], sidx)
            pltpu.sync_copy(dst_hbm.at[:, pl.ds(e, _B)], didx)
            for half in range(2):
                for t in range(0, 64, 16):
                    s64[0, pl.ds(t, 16)] = sidx[0, pl.ds(half * 64 + t, 16)]
                    d64[0, pl.ds(t, 16)] = didx[0, pl.ds(half * 64 + t, 16)]
                pltpu.sync_copy(xl_hbm.at[s64.at[0]], xlb)
                pltpu.sync_copy(xr_hbm.at[d64.at[0]], xrb)

                _edge_body_1h(xlb, xrb, msgb, attr, iota)
                pltpu.sync_copy(msgb, acc.at[d64.at[0]], add=True)

        plsc.subcore_barrier()
        _copy_out(msgb, acc, out_hbm, c, s)

    return k(xl3, xr3, src, dst, att)


def _edge_body_1h(xlb, xrb, msgb, attr, iota):
    @pl.loop(0, 64)
    def _(i):
        xl = [xlb[i, pl.ds(k_ * 16, 16)] for k_ in range(4)]
        acc_v = jnp.zeros((16,), jnp.float32)
        for k4 in range(4):
            ev = xl[k4] + xrb[i, pl.ds(k4 * 16, 16)]
            ea = jnp.maximum(ev, 0.2 * ev)
            acc_v = acc_v + ea * attr[k4]
        alpha = jnp.sum(acc_v)
        exv = jnp.exp(jnp.broadcast_to(alpha, (16,)))
        for k4 in range(4):
            msgb[i, pl.ds(k4 * 16, 16)] = xl[k4] * exv
        msgb[i, pl.ds(64, 16)] = jnp.where(
            iota == 0, exv, jnp.zeros((16,), jnp.float32))


_BLK = 1000


def _tc_layer1(x, wl, wr):
    def body(x_ref, wl_ref, wr_ref, oxl, oxr):
        xv = x_ref[...]
        yl = jnp.dot(xv, wl_ref[...], preferred_element_type=jnp.float32)
        yr = jnp.dot(xv, wr_ref[...], preferred_element_type=jnp.float32)
        oxl[0] = yl[:, :128]
        oxl[1] = yl[:, 128:]
        oxr[0] = yr[:, :128]
        oxr[1] = yr[:, 128:]

    return pl.pallas_call(
        body,
        grid=(N // _BLK,),
        in_specs=[
            pl.BlockSpec((_BLK, 16), lambda i: (i, 0)),
            pl.BlockSpec((16, 256), lambda i: (0, 0)),
            pl.BlockSpec((16, 256), lambda i: (0, 0)),
        ],
        out_specs=[
            pl.BlockSpec((2, _BLK, 128), lambda i: (0, i, 0)),
            pl.BlockSpec((2, _BLK, 128), lambda i: (0, i, 0)),
        ],
        out_shape=[
            jax.ShapeDtypeStruct((2, N, 128), jnp.float32),
            jax.ShapeDtypeStruct((2, N, 128), jnp.float32),
        ],
    )(x, wl, wr)


def _den_reduce(denp):
    def body(d_ref, o_ref):
        o_ref[...] = jnp.sum(d_ref[...], axis=1)

    return pl.pallas_call(
        body,
        out_shape=jax.ShapeDtypeStruct((2, _DROW, 128), jnp.float32),
    )(denp)


def _bn_elu(h, g, be):
    h = g * h / jnp.sqrt(1.0 + 1e-5) + be
    return jnp.where(h > 0, h, jnp.exp(jnp.minimum(h, 0.0)) - 1.0)


def _tc_mid(num, den, bconv, g, be, wl, wr, out_ch):
    halves = out_ch == 256

    def body(num_ref, den_ref, b_ref, g_ref, be_ref, wl_ref, wr_ref,
             oxl, oxr):
        cols = []
        for half in range(2):
            for h in range(2):
                d = den_ref[half, :, h:h + 1]
                cols.append(num_ref[half, :, h * 64:(h + 1) * 64]
                            / (d + 1e-16))
        hv = jnp.concatenate(cols, axis=1) + b_ref[...]
        hv = _bn_elu(hv, g_ref[...], be_ref[...])
        yl = jnp.dot(hv, wl_ref[...], preferred_element_type=jnp.float32)
        yr = jnp.dot(hv, wr_ref[...], preferred_element_type=jnp.float32)
        if halves:
            oxl[0] = yl[:, :128]
            oxl[1] = yl[:, 128:]
            oxr[0] = yr[:, :128]
            oxr[1] = yr[:, 128:]
        else:
            pad = jnp.zeros_like(yl)
            oxl[...] = jnp.concatenate([yl, pad], axis=1)
            oxr[...] = jnp.concatenate([yr, pad], axis=1)

    if halves:
        out_specs = [pl.BlockSpec((2, _BLK, 128), lambda i: (0, i, 0))] * 2
        out_shape = [jax.ShapeDtypeStruct((2, N, 128), jnp.float32)] * 2
    else:
        out_specs = [pl.BlockSpec((_BLK, 128), lambda i: (i, 0))] * 2
        out_shape = [jax.ShapeDtypeStruct((N, 128), jnp.float32)] * 2

    return pl.pallas_call(
        body,
        grid=(N // _BLK,),
        in_specs=[
            pl.BlockSpec((2, _BLK, 128), lambda i: (0, i, 0)),
            pl.BlockSpec((2, _BLK, 2), lambda i: (0, i, 0)),
            pl.BlockSpec((1, 256), lambda i: (0, 0)),
            pl.BlockSpec((1, 256), lambda i: (0, 0)),
            pl.BlockSpec((1, 256), lambda i: (0, 0)),
            pl.BlockSpec((256, out_ch), lambda i: (0, 0)),
            pl.BlockSpec((256, out_ch), lambda i: (0, 0)),
        ],
        out_specs=out_specs,
        out_shape=out_shape,
    )(num, den, bconv, g, be, wl, wr)


def _tc_final(md3, bconv3, g3, be3, batch, cw1, cb1, cw2, cb2):
    def body(md_ref, b_ref, g_ref, be_ref, batch_ref, w1_ref, b1_ref,
             w2_ref, b2_ref, o_ref, pooled_ref):
        md = md_ref[...]
        num = md[0, :, :64] + md[1, :, :64]
        den = md[0, :, 64:65] + md[1, :, 64:65]
        h = num / (den + 1e-16) + b_ref[...]
        h = g_ref[...] * h / jnp.sqrt(1.0 + 1e-5) + be_ref[...]
        bvec = batch_ref[...]

        def per_graph(gi, _):
            mask = bvec == gi
            cnt = jnp.sum(mask.astype(jnp.float32))
            msum = jnp.sum(jnp.where(mask, h, 0.0), axis=0)
            mx = jnp.max(jnp.where(mask, h, -jnp.inf), axis=0)
            mx = jnp.where(jnp.isfinite(mx), mx, 0.0)
            mean = msum / jnp.maximum(cnt, 1.0)
            pooled_ref[pl.ds(gi, 1), :64] = mean[None, :]
            pooled_ref[pl.ds(gi, 1), 64:] = mx[None, :]
            return 0

        lax.fori_loop(0, G, per_graph, 0)
        pooled = pooled_ref[...]
        hc = jnp.maximum(
            jnp.dot(pooled, w1_ref[...], preferred_element_type=jnp.float32)
            + b1_ref[...], 0.0)
        o_ref[...] = (jnp.dot(hc, w2_ref[...],
                              preferred_element_type=jnp.float32)
                      + b2_ref[...])

    return pl.pallas_call(
        body,
        out_shape=jax.ShapeDtypeStruct((G, 1), jnp.float32),
        scratch_shapes=[pltpu.VMEM((G, 128), jnp.float32)],
    )(md3, bconv3, g3, be3, batch, cw1, cb1, cw2, cb2)


def kernel(x, Wl1, Wr1, att1, bconv1, g1, be1, Wl2, Wr2, att2, bconv2, g2,
           be2, Wl3, Wr3, att3, bconv3, g3, be3, cW1, cb1, cW2, cb2,
           edge_index, batch):
    src = edge_index[0:1]
    dst = edge_index[1:2]

    xl1, xr1 = _tc_layer1(x, Wl1, Wr1)
    num1, denp1 = _edge_phase_4h(xl1.reshape(2 * N, 128),
                                 xr1.reshape(2 * N, 128), src, dst, att1)
    den1 = _den_reduce(denp1).reshape(2, _DROW * 64, 2)[:, :N, :]
    xl2, xr2 = _tc_mid(num1, den1, bconv1.reshape(1, 256),
                       g1.reshape(1, 256), be1.reshape(1, 256), Wl2, Wr2, 256)
    num2, denp2 = _edge_phase_4h(xl2.reshape(2 * N, 128),
                                 xr2.reshape(2 * N, 128), src, dst, att2)
    den2 = _den_reduce(denp2).reshape(2, _DROW * 64, 2)[:, :N, :]
    xl3, xr3 = _tc_mid(num2, den2, bconv2.reshape(1, 256),
                       g2.reshape(1, 256), be2.reshape(1, 256), Wl3, Wr3, HID)
    md3 = _edge_phase_1h(xl3, xr3, src, dst, att3)
    out = _tc_final(md3, bconv3.reshape(1, HID), g3.reshape(1, HID),
                    be3.reshape(1, HID), batch.reshape(N, 1),
                    cW1, cb1.reshape(1, HID), cW2, cb2.reshape(1, 1))
    return out
